# baseline (speedup 1.0000x reference)
import functools

import jax
import jax.numpy as jnp
from jax import lax
from jax.experimental import pallas as pl
from jax.experimental.pallas import tpu as pltpu
from jax.experimental.pallas import tpu_sc as plsc

_LANES = 16


def kernel(x, pe_weight):
    batch, seq_len, d_model = x.shape
    pe = pe_weight[:seq_len]
    n_pe = seq_len * d_model
    info = plsc.get_sparse_core_info()
    nw = info.num_cores * info.num_subcores
    per_w = n_pe // nw
    ch = 32768
    n_ch = per_w // ch
    n_vec = ch // _LANES

    mesh = plsc.VectorSubcoreMesh(core_axis_name="c", subcore_axis_name="s")

    @functools.partial(
        pl.kernel,
        out_type=jax.ShapeDtypeStruct((batch * n_pe,), jnp.float32),
        mesh=mesh,
        scratch_types=[
            pltpu.VMEM((ch,), jnp.float32),
            pltpu.VMEM((ch,), jnp.float32),
        ],
    )
    def sc_add(x_hbm, pe_hbm, out_hbm, xbuf, pebuf):
        wid = lax.axis_index("s") * info.num_cores + lax.axis_index("c")
        pbase = wid * per_w

        def chunk_body(j, carry):
            poff = pbase + j * ch
            pltpu.sync_copy(pe_hbm.at[pl.ds(poff, ch)], pebuf)

            def batch_body(b, carry2):
                xoff = b * n_pe + poff
                pltpu.sync_copy(x_hbm.at[pl.ds(xoff, ch)], xbuf)

                def vec_body(i, carry3):
                    sl = pl.ds(i * _LANES, _LANES)
                    xbuf[sl] = xbuf[sl] + pebuf[sl]
                    return carry3

                lax.fori_loop(0, n_vec, vec_body, 0, unroll=8)
                pltpu.sync_copy(xbuf, out_hbm.at[pl.ds(xoff, ch)])
                return carry2

            lax.fori_loop(0, batch, batch_body, 0)
            return carry

        lax.fori_loop(0, n_ch, chunk_body, 0)

    out = sc_add(x.reshape(-1), pe.reshape(-1))
    return out.reshape(batch, seq_len, d_model)


# speedup vs baseline: 1.0303x; 1.0303x over previous
import functools

import jax
import jax.numpy as jnp
from jax import lax
from jax.experimental import pallas as pl
from jax.experimental.pallas import tpu as pltpu
from jax.experimental.pallas import tpu_sc as plsc

_LANES = 16


def kernel(x, pe_weight):
    batch, seq_len, d_model = x.shape
    pe = pe_weight[:seq_len]
    n_pe = seq_len * d_model
    info = plsc.get_sparse_core_info()
    nw = info.num_cores * info.num_subcores
    per_w = n_pe // nw
    ch = 32768
    n_ch = per_w // ch
    n_vec = ch // _LANES
    n_items = n_ch * batch

    mesh = plsc.VectorSubcoreMesh(core_axis_name="c", subcore_axis_name="s")

    @functools.partial(
        pl.kernel,
        out_type=jax.ShapeDtypeStruct((batch * n_pe,), jnp.float32),
        mesh=mesh,
        scratch_types=[
            pltpu.VMEM((2, ch), jnp.float32),
            pltpu.VMEM((ch,), jnp.float32),
            pltpu.SemaphoreType.DMA,
            pltpu.SemaphoreType.DMA,
            pltpu.SemaphoreType.DMA,
            pltpu.SemaphoreType.DMA,
        ],
    )
    def sc_add(x_hbm, pe_hbm, out_hbm, xbufs, pebuf, ld0, ld1, st0, st1):
        wid = lax.axis_index("s") * info.num_cores + lax.axis_index("c")
        pbase = wid * per_w
        ld = (ld0, ld1)
        st = (st0, st1)

        def xoff(t):
            j = t // batch
            b = t % batch
            return b * n_pe + pbase + j * ch

        pltpu.make_async_copy(
            x_hbm.at[pl.ds(xoff(0), ch)], xbufs.at[0], ld0
        ).start()

        def item_body(t, carry):
            kcur = t % 2
            knxt = (t + 1) % 2

            @pl.when(t + 1 < n_items)
            def _():
                @pl.when(t >= 1)
                def _():
                    for k in range(2):
                        @pl.when(knxt == k)
                        def _():
                            pltpu.make_async_copy(
                                xbufs.at[k],
                                out_hbm.at[pl.ds(xoff(t - 1), ch)],
                                st[k],
                            ).wait()

                for k in range(2):
                    @pl.when(knxt == k)
                    def _():
                        pltpu.make_async_copy(
                            x_hbm.at[pl.ds(xoff(t + 1), ch)],
                            xbufs.at[k],
                            ld[k],
                        ).start()

            @pl.when(t % batch == 0)
            def _():
                pltpu.sync_copy(pe_hbm.at[pl.ds(pbase + (t // batch) * ch, ch)], pebuf)

            for k in range(2):
                @pl.when(kcur == k)
                def _():
                    pltpu.make_async_copy(
                        x_hbm.at[pl.ds(xoff(t), ch)], xbufs.at[k], ld[k]
                    ).wait()

                    def vec_body(i, carry3):
                        sl = pl.ds(i * _LANES, _LANES)
                        xbufs.at[k][sl] = xbufs.at[k][sl] + pebuf[sl]
                        return carry3

                    lax.fori_loop(0, n_vec, vec_body, 0, unroll=8)

                    pltpu.make_async_copy(
                        xbufs.at[k],
                        out_hbm.at[pl.ds(xoff(t), ch)],
                        st[k],
                    ).start()

            return carry

        lax.fori_loop(0, n_items, item_body, 0)

        k_last = (n_items - 1) % 2
        pltpu.make_async_copy(
            xbufs.at[k_last],
            out_hbm.at[pl.ds(xoff(n_items - 1), ch)],
            st[k_last],
        ).wait()

    out = sc_add(x.reshape(-1), pe.reshape(-1))
    return out.reshape(batch, seq_len, d_model)


# speedup vs baseline: 7.5644x; 7.3419x over previous
import jax
import jax.numpy as jnp
from jax.experimental import pallas as pl
from jax.experimental.pallas import tpu as pltpu


def _add_pe_kernel(x_ref, pe_ref, o_ref):
    o_ref[...] = x_ref[...] + pe_ref[...][None, :, :]


def kernel(x, pe_weight):
    batch, seq_len, d_model = x.shape
    s_blk = 512
    grid = (seq_len // s_blk,)
    pe = pe_weight[:seq_len]
    return pl.pallas_call(
        _add_pe_kernel,
        grid=grid,
        in_specs=[
            pl.BlockSpec((batch, s_blk, d_model), lambda i: (0, i, 0)),
            pl.BlockSpec((s_blk, d_model), lambda i: (i, 0)),
        ],
        out_specs=pl.BlockSpec((batch, s_blk, d_model), lambda i: (0, i, 0)),
        out_shape=jax.ShapeDtypeStruct((batch, seq_len, d_model), x.dtype),
        compiler_params=pltpu.CompilerParams(
            dimension_semantics=("parallel",),
        ),
    )(x, pe)
